# Initial kernel scaffold; baseline (speedup 1.0000x reference)
import jax
import jax.numpy as jnp
from jax.experimental import pallas as pl


def kernel(x, adj, W, a):
    raise NotImplementedError("write your pallas kernel here")


# speedup vs baseline: 138.0368x; 138.0368x over previous
import functools

import jax
import jax.numpy as jnp
from jax import lax
from jax.experimental import pallas as pl


def _prep_kernel(x_ref, w_ref, a1_ref, a2_ref, wh_ref, s1_ref, s2_ref, cs_ref):
    wh = jnp.dot(x_ref[...], w_ref[...], preferred_element_type=jnp.float32)
    wh_ref[...] = wh
    s1_ref[...] = jnp.dot(wh, a1_ref[...], preferred_element_type=jnp.float32)
    s2_ref[...] = lax.dot_general(
        a2_ref[...], wh, (((0,), (1,)), ((), ())),
        preferred_element_type=jnp.float32)
    cs_ref[...] = jnp.sum(wh, axis=0, keepdims=True)


def _gat_kernel(adj_ref, s1_ref, s2_ref, wh_ref, cs_ref, out_ref, *,
                n_cols, half):
    e = s1_ref[...] + s2_ref[...]
    e = jnp.where(e >= 0.0, e, 0.2 * e)
    e = jnp.where(adj_ref[...] != 0.0, e, 0.0)
    b = jnp.exp(e) - 1.0
    z = jnp.sum(b, axis=1, keepdims=True)
    acc = jnp.dot(b, wh_ref[...], preferred_element_type=jnp.float32)
    h = (cs_ref[...] + acc) / (n_cols + z)
    out_ref[...] = 0.5 * (h[:, :half] + h[:, half:])


def kernel(x, adj, W, a):
    n, _ = x.shape
    nc = adj.shape[1]
    out_f = W.shape[1]
    half = out_f // 2
    a1 = a[:out_f]
    a2 = a[out_f:]

    wh, s1, s2, cs = pl.pallas_call(
        _prep_kernel,
        out_shape=[
            jax.ShapeDtypeStruct((n, out_f), jnp.float32),
            jax.ShapeDtypeStruct((n, 1), jnp.float32),
            jax.ShapeDtypeStruct((1, n), jnp.float32),
            jax.ShapeDtypeStruct((1, out_f), jnp.float32),
        ],
    )(x, W, a1, a2)

    rb = 256
    out = pl.pallas_call(
        functools.partial(_gat_kernel, n_cols=float(nc), half=half),
        grid=(n // rb,),
        in_specs=[
            pl.BlockSpec((rb, nc), lambda i: (i, 0)),
            pl.BlockSpec((rb, 1), lambda i: (i, 0)),
            pl.BlockSpec((1, nc), lambda i: (0, 0)),
            pl.BlockSpec((n, out_f), lambda i: (0, 0)),
            pl.BlockSpec((1, out_f), lambda i: (0, 0)),
        ],
        out_specs=pl.BlockSpec((rb, half), lambda i: (i, 0)),
        out_shape=jax.ShapeDtypeStruct((n, half), jnp.float32),
    )(adj, s1, s2, wh, cs)
    return out


# speedup vs baseline: 149.2499x; 1.0812x over previous
import functools

import jax
import jax.numpy as jnp
from jax import lax
from jax.experimental import pallas as pl


def _prep_kernel(x_ref, w_ref, a1_ref, a2_ref, wh_ref, s1_ref, s2_ref):
    wh = jnp.dot(x_ref[...], w_ref[...], preferred_element_type=jnp.float32)
    wh_ref[...] = wh
    s1_ref[...] = jnp.dot(wh, a1_ref[...], preferred_element_type=jnp.float32)
    s2_ref[...] = lax.dot_general(
        a2_ref[...], wh, (((0,), (1,)), ((), ())),
        preferred_element_type=jnp.float32)


def _gat_kernel(adj_ref, s1_ref, s2_ref, wh_ref, out_ref, *, half):
    e = s1_ref[...] + s2_ref[...]
    e = jnp.maximum(e, 0.2 * e)
    c = jnp.exp(adj_ref[...] * e)
    z = jnp.sum(c, axis=1, keepdims=True)
    acc = jnp.dot(c, wh_ref[...], preferred_element_type=jnp.float32)
    h = acc / z
    out_ref[...] = 0.5 * (h[:, :half] + h[:, half:])


def kernel(x, adj, W, a):
    n, _ = x.shape
    nc = adj.shape[1]
    out_f = W.shape[1]
    half = out_f // 2
    a1 = a[:out_f]
    a2 = a[out_f:]

    wh, s1, s2 = pl.pallas_call(
        _prep_kernel,
        out_shape=[
            jax.ShapeDtypeStruct((n, out_f), jnp.float32),
            jax.ShapeDtypeStruct((n, 1), jnp.float32),
            jax.ShapeDtypeStruct((1, n), jnp.float32),
        ],
    )(x, W, a1, a2)

    rb = 256
    out = pl.pallas_call(
        functools.partial(_gat_kernel, half=half),
        grid=(n // rb,),
        in_specs=[
            pl.BlockSpec((rb, nc), lambda i: (i, 0)),
            pl.BlockSpec((rb, 1), lambda i: (i, 0)),
            pl.BlockSpec((1, nc), lambda i: (0, 0)),
            pl.BlockSpec((n, out_f), lambda i: (0, 0)),
        ],
        out_specs=pl.BlockSpec((rb, half), lambda i: (i, 0)),
        out_shape=jax.ShapeDtypeStruct((n, half), jnp.float32),
    )(adj, s1, s2, wh)
    return out


# speedup vs baseline: 163.2103x; 1.0935x over previous
import functools

import jax
import jax.numpy as jnp
from jax import lax
from jax.experimental import pallas as pl


def _prep_kernel(x_ref, w_ref, a1_ref, a2_ref, wh_ref, s1_ref, s2_ref):
    wh = jnp.dot(x_ref[...], w_ref[...], preferred_element_type=jnp.float32)
    wh_ref[...] = wh
    s1_ref[...] = jnp.dot(wh, a1_ref[...], preferred_element_type=jnp.float32)
    s2_ref[...] = lax.dot_general(
        a2_ref[...], wh, (((0,), (1,)), ((), ())),
        preferred_element_type=jnp.float32)


def _gat_kernel(adj_ref, s1_ref, s2_ref, wh_ref, out_ref, *, half):
    e = s1_ref[...] + s2_ref[...]
    e = jnp.maximum(e, 0.2 * e)
    c = jnp.exp(adj_ref[...] * e)
    z = jnp.sum(c, axis=1, keepdims=True)
    acc = jnp.dot(c, wh_ref[...], preferred_element_type=jnp.float32)
    h = acc / z
    out_ref[...] = 0.5 * (h[:, :half] + h[:, half:])


def kernel(x, adj, W, a):
    n, _ = x.shape
    nc = adj.shape[1]
    out_f = W.shape[1]
    half = out_f // 2
    a1 = a[:out_f]
    a2 = a[out_f:]

    wh, s1, s2 = pl.pallas_call(
        _prep_kernel,
        out_shape=[
            jax.ShapeDtypeStruct((n, out_f), jnp.float32),
            jax.ShapeDtypeStruct((n, 1), jnp.float32),
            jax.ShapeDtypeStruct((1, n), jnp.float32),
        ],
    )(x, W, a1, a2)

    rb = 512
    out = pl.pallas_call(
        functools.partial(_gat_kernel, half=half),
        grid=(n // rb,),
        in_specs=[
            pl.BlockSpec((rb, nc), lambda i: (i, 0)),
            pl.BlockSpec((rb, 1), lambda i: (i, 0)),
            pl.BlockSpec((1, nc), lambda i: (0, 0)),
            pl.BlockSpec((n, out_f), lambda i: (0, 0)),
        ],
        out_specs=pl.BlockSpec((rb, half), lambda i: (i, 0)),
        out_shape=jax.ShapeDtypeStruct((n, half), jnp.float32),
    )(adj, s1, s2, wh)
    return out
